# Initial kernel scaffold; baseline (speedup 1.0000x reference)
import math

import jax
import jax.numpy as jnp
from jax import lax
from jax.experimental import pallas as pl
from jax.experimental.pallas import tpu as pltpu
import jax.experimental.pallas.tpu_sc as plsc

N = 100000
H = 128
B = 256
BD = 272
NC = 2
NS = 16
L = 16
RB = 128
NFULL = N // RB
TAIL = N - NFULL * RB
NBLK = NFULL + 1
KMAX = (NBLK + NC * NS - 1) // (NC * NS)
NEG = -3.0e38


def _sc_body(h_hbm, batch_hbm, stats_out, cnt_out,
             rowbuf, sqbuf, onesbuf, idxbuf, maxacc, mbuf, tbuf,
             zbuf, zbuf16, shared_sum, shared_sq, shared_cnt, maxstage):
  cid = lax.axis_index("c")
  sid = lax.axis_index("s")
  wid = sid * NC + cid
  iota = lax.iota(jnp.int32, L)

  def fill2d(ref, nrows, val):
    v = jnp.full((L,), val, jnp.float32)
    def body(i, _):
      ref[i // 8, pl.ds((i % 8) * L, L)] = v
      return 0
    lax.fori_loop(0, nrows * 8, body, 0)

  fill2d(maxacc, BD, NEG)
  fill2d(zbuf, 17, 0.0)

  def fill16(i, _):
    onesbuf[i] = jnp.full((L,), 1.0, jnp.float32)
    zbuf16[i // 8] = jnp.full((L,), 0.0, jnp.float32)
    return 0
  lax.fori_loop(0, RB, fill16, 0)

  base = sid * 17
  pltpu.sync_copy(zbuf, shared_sum.at[pl.ds(base, 17)])
  pltpu.sync_copy(zbuf, shared_sq.at[pl.ds(base, 17)])
  pltpu.sync_copy(zbuf16, shared_cnt.at[pl.ds(base, 17)])
  plsc.subcore_barrier()

  def blk_body(k, carry):
    blk = wid + (NC * NS) * k
    in_range = blk < NBLK
    is_last = blk == NFULL

    @pl.when(in_range & jnp.logical_not(is_last))
    def _():
      pltpu.sync_copy(h_hbm.at[pl.ds(blk * RB, RB)], rowbuf)
      pltpu.sync_copy(batch_hbm.at[pl.ds(blk * RB, RB)], idxbuf)

    @pl.when(is_last)
    def _():
      pltpu.sync_copy(h_hbm.at[pl.ds(NFULL * RB, TAIL)],
                      rowbuf.at[pl.ds(0, TAIL)])
      pltpu.sync_copy(batch_hbm.at[pl.ds(NFULL * RB, TAIL)],
                      idxbuf.at[pl.ds(0, TAIL)])
      idxbuf[pl.ds(TAIL, L)] = jnp.full((L,), B, jnp.int32)
      idxbuf[pl.ds(TAIL + L, L)] = jnp.full((L,), B, jnp.int32)

    @pl.when(in_range)
    def _():
      pltpu.sync_copy(rowbuf, shared_sum.at[idxbuf], add=True)

    def row_body(r, rc):
      prev = rc[0]
      runs = rc[1:]
      ids = plsc.load_gather(idxbuf, [jnp.full((L,), r, jnp.int32)])
      flushp = jnp.any((prev >= 0) & (ids != prev))

      @pl.when(flushp)
      def _():
        for g in range(8):
          lane = iota + g * L
          cur = plsc.load_gather(maxacc, [prev, lane])
          plsc.store_scatter(maxacc, [prev, lane],
                             jnp.maximum(cur, runs[g]))

      same = ids == prev
      newruns = []
      for g in range(8):
        v = rowbuf[r, pl.ds(g * L, L)]
        sqbuf[r, pl.ds(g * L, L)] = v * v
        newruns.append(jnp.where(same, jnp.maximum(runs[g], v), v))
      return (ids,) + tuple(newruns)

    carry = lax.fori_loop(0, RB, row_body, carry)

    @pl.when(in_range)
    def _():
      pltpu.sync_copy(sqbuf, shared_sq.at[idxbuf], add=True)
      pltpu.sync_copy(onesbuf, shared_cnt.at[idxbuf], add=True)
    return carry

  init = (jnp.full((L,), -1, jnp.int32),) + tuple(
      jnp.full((L,), NEG, jnp.float32) for _ in range(8))
  carry = lax.fori_loop(0, KMAX, blk_body, init)

  prev = carry[0]
  runs = carry[1:]

  @pl.when(jnp.any(prev >= 0))
  def _():
    for g in range(8):
      lane = iota + g * L
      cur = plsc.load_gather(maxacc, [prev, lane])
      plsc.store_scatter(maxacc, [prev, lane], jnp.maximum(cur, runs[g]))

  plsc.subcore_barrier()

  pltpu.sync_copy(maxacc, maxstage.at[sid])
  plsc.subcore_barrier()

  pltpu.sync_copy(maxstage.at[0, pl.ds(base, 17)], mbuf)
  for i in range(1, NS):
    pltpu.sync_copy(maxstage.at[i, pl.ds(base, 17)], tbuf)
    def mrg(j, _):
      for g in range(8):
        s = pl.ds(g * L, L)
        mbuf[j, s] = jnp.maximum(mbuf[j, s], tbuf[j, s])
      return 0
    lax.fori_loop(0, 17, mrg, 0)

  pltpu.sync_copy(mbuf, stats_out.at[cid, 2, pl.ds(base, 17)])
  pltpu.sync_copy(shared_sum.at[pl.ds(base, 17)],
                  stats_out.at[cid, 0, pl.ds(base, 17)])
  pltpu.sync_copy(shared_sq.at[pl.ds(base, 17)],
                  stats_out.at[cid, 1, pl.ds(base, 17)])
  pltpu.sync_copy(shared_cnt.at[pl.ds(base, 17)],
                  cnt_out.at[cid, pl.ds(base, 17)])


def _pool_sc(h, batch):
  mesh = plsc.VectorSubcoreMesh(core_axis_name="c", subcore_axis_name="s",
                                num_cores=NC, num_subcores=NS)
  f = pl.kernel(
      _sc_body,
      out_type=[
          jax.ShapeDtypeStruct((NC, 3, BD, H), jnp.float32),
          jax.ShapeDtypeStruct((NC, BD, L), jnp.float32),
      ],
      mesh=mesh,
      scratch_types=[
          pltpu.VMEM((RB, H), jnp.float32),
          pltpu.VMEM((RB, H), jnp.float32),
          pltpu.VMEM((RB, L), jnp.float32),
          pltpu.VMEM((RB,), jnp.int32),
          pltpu.VMEM((BD, H), jnp.float32),
          pltpu.VMEM((17, H), jnp.float32),
          pltpu.VMEM((17, H), jnp.float32),
          pltpu.VMEM((17, H), jnp.float32),
          pltpu.VMEM((17, L), jnp.float32),
          pltpu.VMEM_SHARED((BD, H), jnp.float32),
          pltpu.VMEM_SHARED((BD, H), jnp.float32),
          pltpu.VMEM_SHARED((BD, L), jnp.float32),
          pltpu.VMEM_SHARED((NS, BD, H), jnp.float32),
      ],
  )
  return f(h, batch)


def _tc_body(stats_ref, cnt_ref, w1_ref, b1_ref, w2_ref, b2_ref, out_ref):
  st = stats_ref[...]
  cn = cnt_ref[...]
  ssum = st[0, 0, :B, :] + st[1, 0, :B, :]
  ssq = st[0, 1, :B, :] + st[1, 1, :B, :]
  smax = jnp.maximum(st[0, 2, :B, :], st[1, 2, :B, :])
  count = cn[0, :B, 0] + cn[1, :B, 0]
  safe = jnp.maximum(count, 1.0)[:, None]
  mean = ssum / safe
  var = jnp.maximum(ssq / safe - mean * mean, 0.0)
  std = jnp.sqrt(var + 1e-8)
  smax = jnp.where(count[:, None] > 0.0, smax, 0.0)
  g = jnp.concatenate([mean, smax, std], axis=1)
  hid = jax.nn.relu(
      jnp.dot(g, w1_ref[...], preferred_element_type=jnp.float32)
      + b1_ref[...])
  z = jnp.tanh(
      jnp.dot(hid, w2_ref[...], preferred_element_type=jnp.float32)
      + b2_ref[...]) * math.pi
  out_ref[...] = z


def _head_tc(stats, cnt, W1, b1, W2, b2):
  w2p = jnp.zeros((32, 128), jnp.float32).at[:, :8].set(W2)
  b2p = jnp.zeros((1, 128), jnp.float32).at[:, :8].set(b2)
  out = pl.pallas_call(
      _tc_body,
      out_shape=jax.ShapeDtypeStruct((B, 128), jnp.float32),
  )(stats, cnt, W1, b1.reshape(1, 32), w2p, b2p)
  return out[:, :8]


def kernel(h, batch, W1, b1, W2, b2):
  stats, cnt = _pool_sc(h, batch)
  return _head_tc(stats, cnt, W1, b1, W2, b2)


# speedup vs baseline: 3.3093x; 3.3093x over previous
import math

import jax
import jax.numpy as jnp
from jax import lax
from jax.experimental import pallas as pl
from jax.experimental.pallas import tpu as pltpu
import jax.experimental.pallas.tpu_sc as plsc

H = 128
HC = 64
B = 256
BD = 272
NC = 2
NS = 16
L = 16
RB = 128
NG = HC // L
NEG = -3.0e38


def _make_sc_body(n):
  nfull = n // RB
  tail = n - nfull * RB
  nblk = nfull + (1 if tail else 0)
  kmax = (nblk + NS - 1) // NS
  assert tail % L == 0

  def _sc_body(h2_hbm, batch_hbm, stats_out,
               rowbuf, idxg, sumacc, sqacc, maxacc):
    cid = lax.axis_index("c")
    sid = lax.axis_index("s")
    iota = lax.iota(jnp.int32, L)

    def fill2d(ref, nrows, val):
      v = jnp.full((L,), val, jnp.float32)
      def body(i, _):
        ref[i // 8, pl.ds((i % 8) * L, L)] = v
        return 0
      lax.fori_loop(0, nrows * 8, body, 0)

    fill2d(maxacc, BD, NEG)
    fill2d(sumacc, BD, 0.0)
    fill2d(sqacc, BD, 0.0)

    def flush(prev, rcnt, rsum, rsq, rmax):
      cs = pl.ds(HC, L)
      sumacc[prev, cs] = sumacc[prev, cs] + rcnt
      for g in range(NG):
        sl = pl.ds(g * L, L)
        sumacc[prev, sl] = sumacc[prev, sl] + rsum[g]
        sqacc[prev, sl] = sqacc[prev, sl] + rsq[g]
        maxacc[prev, sl] = jnp.maximum(maxacc[prev, sl], rmax[g])

    zeroN = tuple(jnp.zeros((L,), jnp.float32) for _ in range(NG))
    negN = tuple(jnp.full((L,), NEG, jnp.float32) for _ in range(NG))

    def blk_body(k, carry):
      blk = sid + NS * k
      in_range = blk < nblk
      is_last = blk == (nblk - 1) if tail else jnp.bool_(False)

      @pl.when(in_range & jnp.logical_not(is_last))
      def _():
        pltpu.sync_copy(h2_hbm.at[pl.ds(blk * RB, RB)], rowbuf)
        pltpu.sync_copy(batch_hbm.at[pl.ds(blk * RB, RB)],
                        idxg.at[pl.ds(0, RB)])

      if tail:
        @pl.when(is_last)
        def _():
          pltpu.sync_copy(h2_hbm.at[pl.ds(nfull * RB, tail)],
                          rowbuf.at[pl.ds(0, tail)])
          pltpu.sync_copy(batch_hbm.at[pl.ds(nfull * RB, tail)],
                          idxg.at[pl.ds(0, tail)])
          for off in range(tail, RB, L):
            idxg[pl.ds(off, L)] = jnp.full((L,), B, jnp.int32)

      @pl.when(jnp.logical_not(in_range))
      def _():
        def dfill(i, _):
          idxg[pl.ds(i * L, L)] = jnp.full((L,), B, jnp.int32)
          return 0
        lax.fori_loop(0, (RB + L) // L, dfill, 0)

      def row_body(r, rc):
        prev = rc[0]
        rcnt = rc[1]
        rsum = rc[2:2 + NG]
        rsq = rc[2 + NG:2 + 2 * NG]
        rmax = rc[2 + 2 * NG:2 + 3 * NG]
        s = idxg[pl.ds(r, L)][0]
        changed = s != prev

        @pl.when((prev >= 0) & changed)
        def _():
          flush(prev, rcnt, rsum, rsq, rmax)

        st = lax.cond(
            changed,
            lambda: (jnp.zeros((L,), jnp.float32),) + zeroN + zeroN + negN,
            lambda: (rcnt,) + tuple(rsum) + tuple(rsq) + tuple(rmax))
        ncnt = st[0] + 1.0
        nsum, nsq, nmax = [], [], []
        for g in range(NG):
          v = rowbuf[r, pl.ds(cid * HC + g * L, L)]
          nsum.append(st[1 + g] + v)
          nsq.append(st[1 + NG + g] + v * v)
          nmax.append(jnp.maximum(st[1 + 2 * NG + g], v))
        return (s, ncnt) + tuple(nsum) + tuple(nsq) + tuple(nmax)

      return lax.fori_loop(0, RB, row_body, carry)

    init = (jnp.int32(-1), jnp.zeros((L,), jnp.float32)) + zeroN + zeroN + negN
    carry = lax.fori_loop(0, kmax, blk_body, init)

    @pl.when(carry[0] >= 0)
    def _():
      flush(carry[0], carry[1], carry[2:2 + NG], carry[2 + NG:2 + 2 * NG],
            carry[2 + 2 * NG:2 + 3 * NG])

    pltpu.sync_copy(sumacc.at[pl.ds(0, B)], stats_out.at[cid, sid, 0])
    pltpu.sync_copy(sqacc.at[pl.ds(0, B)], stats_out.at[cid, sid, 1])
    pltpu.sync_copy(maxacc.at[pl.ds(0, B)], stats_out.at[cid, sid, 2])

  return _sc_body


def _pool_sc(h, batch, interpret=False):
  n = h.shape[0]
  mesh = plsc.VectorSubcoreMesh(core_axis_name="c", subcore_axis_name="s",
                                num_cores=NC, num_subcores=NS)
  f = pl.kernel(
      _make_sc_body(n),
      out_type=[
          jax.ShapeDtypeStruct((NC, NS, 3, B, H), jnp.float32),
      ],
      mesh=mesh,
      interpret=interpret,
      scratch_types=[
          pltpu.VMEM((RB, H), jnp.float32),
          pltpu.VMEM((RB + L,), jnp.int32),
          pltpu.VMEM((BD, H), jnp.float32),
          pltpu.VMEM((BD, H), jnp.float32),
          pltpu.VMEM((BD, H), jnp.float32),
      ],
  )
  return f(h, batch)[0]


def _tc_body(stats_ref, w1_ref, b1_ref, w2_ref, b2_ref, out_ref):
  st = stats_ref[...]
  s0 = jnp.sum(st[0, :, 0], axis=0)
  s1 = jnp.sum(st[1, :, 0], axis=0)
  q0 = jnp.sum(st[0, :, 1], axis=0)
  q1 = jnp.sum(st[1, :, 1], axis=0)
  m0 = jnp.max(st[0, :, 2], axis=0)
  m1 = jnp.max(st[1, :, 2], axis=0)
  ssum = jnp.concatenate([s0[:, :HC], s1[:, :HC]], axis=1)
  ssq = jnp.concatenate([q0[:, :HC], q1[:, :HC]], axis=1)
  smax = jnp.concatenate([m0[:, :HC], m1[:, :HC]], axis=1)
  count = s0[:, HC]
  safe = jnp.maximum(count, 1.0)[:, None]
  mean = ssum / safe
  var = jnp.maximum(ssq / safe - mean * mean, 0.0)
  std = jnp.sqrt(var + 1e-8)
  smax = jnp.where(count[:, None] > 0.0, smax, 0.0)
  g = jnp.concatenate([mean, smax, std], axis=1)
  hid = jax.nn.relu(
      jnp.dot(g, w1_ref[...], preferred_element_type=jnp.float32)
      + b1_ref[...])
  z = jnp.tanh(
      jnp.dot(hid, w2_ref[...], preferred_element_type=jnp.float32)
      + b2_ref[...]) * math.pi
  out_ref[...] = z


def _head_tc(stats, W1, b1, W2, b2, interpret=False):
  w2p = jnp.zeros((32, 128), jnp.float32).at[:, :8].set(W2)
  b2p = jnp.zeros((1, 128), jnp.float32).at[:, :8].set(b2)
  out = pl.pallas_call(
      _tc_body,
      out_shape=jax.ShapeDtypeStruct((B, 128), jnp.float32),
      interpret=interpret,
  )(stats, W1, b1.reshape(1, 32), w2p, b2p)
  return out[:, :8]


def kernel(h, batch, W1, b1, W2, b2):
  stats = _pool_sc(h, batch)
  return _head_tc(stats, W1, b1, W2, b2)
